# Initial kernel scaffold; baseline (speedup 1.0000x reference)
import jax
import jax.numpy as jnp
from jax.experimental import pallas as pl


def kernel(out, target, ind, mask):
    raise NotImplementedError("write your pallas kernel here")


# speedup vs baseline: 1.2361x; 1.2361x over previous
import functools

import jax
import jax.numpy as jnp
from jax import lax
from jax.experimental import pallas as pl
from jax.experimental.pallas import tpu as pltpu
from jax.experimental.pallas import tpu_sc as plsc

NC, NS, L = 2, 16, 16
NW = NC * NS
B, N, S = 64, 128, 2
HW = 128 * 128
BPW = B // NW
NCHUNK = N // L


def _sc_partials(out_flat, ind, mask, target_flat):
    mesh = plsc.VectorSubcoreMesh(
        core_axis_name="c", subcore_axis_name="s",
        num_cores=NC, num_subcores=NS)

    @functools.partial(
        pl.kernel,
        out_type=[jax.ShapeDtypeStruct((NW, L), jnp.float32),
                  jax.ShapeDtypeStruct((NW, L), jnp.float32)],
        mesh=mesh,
        scratch_types=[
            pltpu.VMEM((N,), jnp.int32),
            pltpu.VMEM((N,), jnp.int32),
            pltpu.VMEM((N,), jnp.int32),
            pltpu.VMEM((N,), jnp.float32),
            pltpu.VMEM((N,), jnp.float32),
            pltpu.VMEM((N,), jnp.float32),
            pltpu.VMEM((N,), jnp.float32),
            pltpu.VMEM((N,), jnp.float32),
            pltpu.VMEM((L,), jnp.float32),
            pltpu.VMEM((L,), jnp.float32),
            pltpu.SemaphoreType.DMA,
        ],
    )
    def k(out_hbm, ind_hbm, mask_hbm, tgt_hbm, loss_hbm, msum_hbm,
          ind_v, idx0_v, idx1_v, p0_v, p1_v, t0_v, t1_v, m_v, lo_v, mo_v, sem):
        wid = lax.axis_index("s") * NC + lax.axis_index("c")
        lacc = jnp.zeros((L,), jnp.float32)
        macc = jnp.zeros((L,), jnp.float32)
        for j in range(BPW):
            b = wid * BPW + j
            pltpu.sync_copy(ind_hbm.at[b], ind_v)
            pltpu.sync_copy(mask_hbm.at[b], m_v)
            pltpu.sync_copy(tgt_hbm.at[0, b], t0_v)
            pltpu.sync_copy(tgt_hbm.at[1, b], t1_v)
            base0 = b * (S * HW)
            for i in range(NCHUNK):
                c = ind_v[pl.ds(i * L, L)]
                idx0_v[pl.ds(i * L, L)] = c + base0
                idx1_v[pl.ds(i * L, L)] = c + (base0 + HW)
            cp0 = pltpu.async_copy(out_hbm.at[idx0_v], p0_v, sem)
            cp1 = pltpu.async_copy(out_hbm.at[idx1_v], p1_v, sem)
            cp0.wait()
            cp1.wait()
            for i in range(NCHUNK):
                m = m_v[pl.ds(i * L, L)]
                p0 = p0_v[pl.ds(i * L, L)]
                p1 = p1_v[pl.ds(i * L, L)]
                t0 = t0_v[pl.ds(i * L, L)]
                t1 = t1_v[pl.ds(i * L, L)]
                lacc = lacc + jnp.abs(p0 * m - t0 * m) + jnp.abs(p1 * m - t1 * m)
                macc = macc + m
        lo_v[...] = lacc
        mo_v[...] = macc
        pltpu.sync_copy(lo_v, loss_hbm.at[wid])
        pltpu.sync_copy(mo_v, msum_hbm.at[wid])

    return k(out_flat, ind, mask, target_flat)


def _finalize(loss_parts, mask_parts):
    def k(l_ref, m_ref, o_ref):
        num = jnp.sum(l_ref[...], keepdims=True)
        den = jnp.sum(m_ref[...], keepdims=True) + 0.0001
        o_ref[...] = num / den

    r = pl.pallas_call(
        k, out_shape=jax.ShapeDtypeStruct((1, 1), jnp.float32),
    )(loss_parts, mask_parts)
    return r[0, 0]


def kernel(out, target, ind, mask):
    out_flat = out.reshape(-1)
    tgt_planes = jnp.moveaxis(target, 2, 0)
    loss_parts, mask_parts = _sc_partials(out_flat, ind, mask, tgt_planes)
    return _finalize(loss_parts, mask_parts)
